# Initial kernel scaffold; baseline (speedup 1.0000x reference)
import jax
import jax.numpy as jnp
from jax.experimental import pallas as pl


def kernel(x, edge_index, W_in, b_in, W_conv, b_conv, W_out, b_out):
    raise NotImplementedError("write your pallas kernel here")


# speedup vs baseline: 5.0648x; 5.0648x over previous
import functools

import jax
import jax.numpy as jnp
from jax import lax
from jax.experimental import pallas as pl
from jax.experimental.pallas import tpu as pltpu
from jax.experimental.pallas import tpu_sc as plsc

NC = 2
NS = 16
NW = NC * NS
LANES = 16
CHUNK = 128


def _tc_tables(x_ref, win_ref, bin_ref, w1h_ref, w1x_ref, bc_ref,
               w2h_ref, w2x_ref, a_ref, b_ref):
    x = x_ref[...]
    h = jnp.tanh(
        jnp.dot(x, win_ref[...], preferred_element_type=jnp.float32)
        + bin_ref[...])
    a_ref[...] = (
        jnp.dot(h, w1h_ref[...], preferred_element_type=jnp.float32)
        + jnp.dot(x, w1x_ref[...], preferred_element_type=jnp.float32)
        + bc_ref[...])
    b_ref[...] = (
        jnp.dot(h, w2h_ref[...], preferred_element_type=jnp.float32)
        + jnp.dot(x, w2x_ref[...], preferred_element_type=jnp.float32))


def _make_sc_edge_sum(n_pad, d, epw):
    chunks = epw // CHUNK
    nvec = d // LANES
    mesh = plsc.VectorSubcoreMesh(core_axis_name="c", subcore_axis_name="s")

    @functools.partial(
        pl.kernel, mesh=mesh,
        out_type=jax.ShapeDtypeStruct((NW, d), jnp.float32),
        scratch_types=[
            pltpu.VMEM((epw,), jnp.int32),
            pltpu.VMEM((epw,), jnp.int32),
            pltpu.VMEM((CHUNK, d), jnp.float32),
            pltpu.VMEM((CHUNK, d), jnp.float32),
            pltpu.VMEM((d,), jnp.float32),
            pltpu.SemaphoreType.DMA,
        ],
    )
    def sc_edge_sum(a_hbm, b_hbm, dst_hbm, src_hbm, out_hbm,
                    dsti, srci, arows, brows, accv, sem):
        wid = lax.axis_index("s") * NC + lax.axis_index("c")
        base = wid * epw
        pltpu.sync_copy(dst_hbm.at[pl.ds(base, epw)], dsti)
        pltpu.sync_copy(src_hbm.at[pl.ds(base, epw)], srci)

        def chunk_body(cidx, accs):
            off = cidx * CHUNK
            cp_a = pltpu.async_copy(a_hbm.at[dsti.at[pl.ds(off, CHUNK)]],
                                    arows, sem)
            cp_b = pltpu.async_copy(b_hbm.at[srci.at[pl.ds(off, CHUNK)]],
                                    brows, sem)
            cp_a.wait()
            cp_b.wait()

            def edge_body(i, acc):
                new = []
                for j in range(nvec):
                    va = arows[i, pl.ds(LANES * j, LANES)]
                    vb = brows[i, pl.ds(LANES * j, LANES)]
                    z = va + vb
                    s = 1.0 / (1.0 + jnp.exp(-z))
                    new.append(acc[j] + s)
                return tuple(new)

            return lax.fori_loop(0, CHUNK, edge_body, accs)

        accs0 = tuple(jnp.zeros((LANES,), jnp.float32) for _ in range(nvec))
        accs = lax.fori_loop(0, chunks, chunk_body, accs0)
        for j in range(nvec):
            accv[pl.ds(LANES * j, LANES)] = accs[j]
        pltpu.sync_copy(accv, out_hbm.at[wid])

    return sc_edge_sum


def kernel(x, edge_index, W_in, b_in, W_conv, b_conv, W_out, b_out):
    n, d = x.shape
    hd = W_in.shape[1]
    e = edge_index.shape[1]

    w1 = W_conv[:hd + d] - W_conv[hd + d:]
    wb = W_conv[hd + d:]

    a_tab, b_tab = pl.pallas_call(
        _tc_tables,
        out_shape=[
            jax.ShapeDtypeStruct((n, hd), jnp.float32),
            jax.ShapeDtypeStruct((n, hd), jnp.float32),
        ],
    )(x, W_in, b_in.reshape(1, -1), w1[:hd], w1[hd:],
      b_conv.reshape(1, -1), wb[:hd], wb[hd:])

    n_pad = n + 8
    a_pad = jnp.concatenate(
        [a_tab, jnp.full((n_pad - n, hd), -60.0, jnp.float32)], axis=0)
    b_pad = jnp.concatenate(
        [b_tab, jnp.zeros((n_pad - n, hd), jnp.float32)], axis=0)

    src = edge_index[0]
    dst = edge_index[1]
    grain = NW * CHUNK
    e_pad = ((e + grain - 1) // grain) * grain
    fill = jnp.full((e_pad - e,), n, jnp.int32)
    dst_p = jnp.concatenate([dst, fill])
    src_p = jnp.concatenate([src, fill])

    per_w = _make_sc_edge_sum(n_pad, hd, e_pad // NW)(
        a_pad, b_pad, dst_p, src_p)

    s = per_w.sum(axis=0) / n
    return jax.nn.sigmoid(s @ W_out + b_out)
